# baseline (speedup 1.0000x reference)
import functools

import jax
import jax.numpy as jnp
from jax import lax
from jax.experimental import pallas as pl
from jax.experimental.pallas import tpu as pltpu
from jax.experimental.pallas import tpu_sc as plsc

NC = 2
NS = 16
L = 16
NW = NC * NS

RB = 256


def _compute_block(in_ref, out_ref, carry):
    g = len(carry)

    def row(r, c):
        new = []
        for j in range(g):
            v = in_ref[r, pl.ds(j * L, L)]
            acc = c[j] + v
            out_ref[r, pl.ds(j * L, L)] = acc
            new.append(acc)
        return tuple(new)

    return plsc.parallel_loop(0, RB, unroll=4, carry=tuple(carry))(row)


def _sc_body(x_hbm, o_hbm, in0, in1, out0, out1, ls0, ls1, ss0, ss1):
    b_, n, c = x_hbm.shape
    cw = c // NW
    g = cw // L
    nblk = n // RB
    wid = lax.axis_index("s") * NC + lax.axis_index("c")
    c0 = wid * cw

    ins = [in0, in1]
    outs = [out0, out1]
    lsems = [ls0, ls1]
    ssems = [ss0, ss1]

    def load(k, b, j):
        src = x_hbm.at[b, pl.ds(j * RB, RB), pl.ds(c0, cw)]
        pltpu.make_async_copy(src, ins[k], lsems[k]).start()

    def store(k, b, j):
        dst = o_hbm.at[b, pl.ds(j * RB, RB), pl.ds(c0, cw)]
        pltpu.make_async_copy(outs[k], dst, ssems[k]).start()

    def wait_load(k):
        pltpu.make_async_copy(x_hbm.at[0, pl.ds(0, RB), pl.ds(0, cw)], ins[k],
                              lsems[k]).wait()

    def wait_store(k):
        pltpu.make_async_copy(outs[k], o_hbm.at[0, pl.ds(0, RB), pl.ds(0, cw)],
                              ssems[k]).wait()

    for b in range(b_):
        load(0, b, 0)
        load(1, b, 1)
        carry = [jnp.zeros((L,), jnp.float32)] * g
        for j in range(nblk):
            k = j % 2
            wait_load(k)
            if b > 0 or j >= 2:
                wait_store(k)
            carry = _compute_block(ins[k], outs[k], carry)
            store(k, b, j)
            if j + 2 < nblk:
                load(k, b, j + 2)
    wait_store(0)
    wait_store(1)


def kernel(x):
    b, n, c = x.shape
    f = pl.kernel(
        _sc_body,
        out_type=jax.ShapeDtypeStruct((b, n, c), x.dtype),
        mesh=plsc.VectorSubcoreMesh(core_axis_name="c", subcore_axis_name="s"),
        scratch_types=[
            pltpu.VMEM((RB, c // NW), jnp.float32),
            pltpu.VMEM((RB, c // NW), jnp.float32),
            pltpu.VMEM((RB, c // NW), jnp.float32),
            pltpu.VMEM((RB, c // NW), jnp.float32),
            pltpu.SemaphoreType.DMA,
            pltpu.SemaphoreType.DMA,
            pltpu.SemaphoreType.DMA,
            pltpu.SemaphoreType.DMA,
        ],
    )
    return f(x)


# speedup vs baseline: 1.0011x; 1.0011x over previous
import functools

import jax
import jax.numpy as jnp
from jax import lax
from jax.experimental import pallas as pl
from jax.experimental.pallas import tpu as pltpu
from jax.experimental.pallas import tpu_sc as plsc

NC = 2
NS = 16
L = 16
NW = NC * NS

RB = 256


def _compute_block(in_ref, out_ref, carry):
    g = len(carry)

    def row(r, c):
        new = []
        for j in range(g):
            v = in_ref[r, pl.ds(j * L, L)]
            acc = c[j] + v
            out_ref[r, pl.ds(j * L, L)] = acc
            new.append(acc)
        return tuple(new)

    return plsc.parallel_loop(0, RB, unroll=8, carry=tuple(carry))(row)


def _sc_body(x_hbm, o_hbm, in0, in1, out0, out1, ls0, ls1, ss0, ss1):
    b_, n, c = x_hbm.shape
    cw = c // NW
    g = cw // L
    nblk = n // RB
    wid = lax.axis_index("s") * NC + lax.axis_index("c")
    c0 = wid * cw

    ins = [in0, in1]
    outs = [out0, out1]
    lsems = [ls0, ls1]
    ssems = [ss0, ss1]

    def load(k, b, j):
        src = x_hbm.at[b, pl.ds(j * RB, RB), pl.ds(c0, cw)]
        pltpu.make_async_copy(src, ins[k], lsems[k]).start()

    def store(k, b, j):
        dst = o_hbm.at[b, pl.ds(j * RB, RB), pl.ds(c0, cw)]
        pltpu.make_async_copy(outs[k], dst, ssems[k]).start()

    def wait_load(k):
        pltpu.make_async_copy(x_hbm.at[0, pl.ds(0, RB), pl.ds(0, cw)], ins[k],
                              lsems[k]).wait()

    def wait_store(k):
        pltpu.make_async_copy(outs[k], o_hbm.at[0, pl.ds(0, RB), pl.ds(0, cw)],
                              ssems[k]).wait()

    for b in range(b_):
        load(0, b, 0)
        load(1, b, 1)
        carry = [jnp.zeros((L,), jnp.float32)] * g
        for j in range(nblk):
            k = j % 2
            wait_load(k)
            if b > 0 or j >= 2:
                wait_store(k)
            carry = _compute_block(ins[k], outs[k], carry)
            store(k, b, j)
            if j + 2 < nblk:
                load(k, b, j + 2)
    wait_store(0)
    wait_store(1)


def kernel(x):
    b, n, c = x.shape
    f = pl.kernel(
        _sc_body,
        out_type=jax.ShapeDtypeStruct((b, n, c), x.dtype),
        mesh=plsc.VectorSubcoreMesh(core_axis_name="c", subcore_axis_name="s"),
        scratch_types=[
            pltpu.VMEM((RB, c // NW), jnp.float32),
            pltpu.VMEM((RB, c // NW), jnp.float32),
            pltpu.VMEM((RB, c // NW), jnp.float32),
            pltpu.VMEM((RB, c // NW), jnp.float32),
            pltpu.SemaphoreType.DMA,
            pltpu.SemaphoreType.DMA,
            pltpu.SemaphoreType.DMA,
            pltpu.SemaphoreType.DMA,
        ],
    )
    return f(x)
